# Initial kernel scaffold; baseline (speedup 1.0000x reference)
import jax
import jax.numpy as jnp
from jax.experimental import pallas as pl


def kernel(hidden_states, Wg, W1, b1, W2, b2):
    raise NotImplementedError("write your pallas kernel here")


# speedup vs baseline: 3.5130x; 3.5130x over previous
import functools
import math

import jax
import jax.numpy as jnp
from jax.experimental import pallas as pl
from jax.experimental.pallas import tpu as pltpu

_E = 8
_TOPK = 2
_FT = 256

_INV_SQRT2 = 1.0 / math.sqrt(2.0)


def _gelu_exact(h):
    return 0.5 * h * (1.0 + jax.lax.erf(h * _INV_SQRT2))


def _router_kernel(x_ref, wg_ref, sel_ref, rw_ref):
    x = x_ref[...]
    wg = wg_ref[...]
    logits = jax.lax.dot_general(
        x.astype(jnp.bfloat16), wg.astype(jnp.bfloat16),
        (((1,), (1,)), ((), ())), preferred_element_type=jnp.float32)
    s = jnp.sum(logits, axis=0, keepdims=True)
    eiota = jax.lax.broadcasted_iota(jnp.int32, (1, _E), 1)
    m0 = jnp.max(s)
    i0 = jnp.min(jnp.where(s == m0, eiota, _E))
    s1 = jnp.where(eiota == i0, -jnp.inf, s)
    m1 = jnp.max(s1)
    i1 = jnp.min(jnp.where(s1 == m1, eiota, _E))
    sel_ref[0] = i0
    sel_ref[1] = i1
    l0 = jnp.sum(jnp.where(eiota == i0, logits, 0.0), axis=1, keepdims=True)
    l1 = jnp.sum(jnp.where(eiota == i1, logits, 0.0), axis=1, keepdims=True)
    m = jnp.maximum(l0, l1)
    e0 = jnp.exp(l0 - m)
    e1 = jnp.exp(l1 - m)
    denom = e0 + e1
    r0 = e0 / denom
    r1 = e1 / denom
    lane = jax.lax.broadcasted_iota(jnp.int32, rw_ref.shape, 1)
    rw_ref[...] = jnp.where(lane == 0, r0, jnp.where(lane == 1, r1, 0.0))


def _expert_kernel(sel_ref, x_ref, rw_ref, w1a_ref, w1b_ref, w2a_ref,
                   w2b_ref, b1a_ref, b1b_ref, b2a_ref, b2b_ref, out_ref):
    nf = pl.program_id(0)
    x = x_ref[...].astype(jnp.bfloat16)
    rw0 = rw_ref[:, 0:1]
    rw1 = rw_ref[:, 1:2]

    h0 = jax.lax.dot_general(
        x, w1a_ref[0].astype(jnp.bfloat16),
        (((1,), (1,)), ((), ())), preferred_element_type=jnp.float32)
    h0 = _gelu_exact(h0 + b1a_ref[0]) * rw0
    h1 = jax.lax.dot_general(
        x, w1b_ref[0].astype(jnp.bfloat16),
        (((1,), (1,)), ((), ())), preferred_element_type=jnp.float32)
    h1 = _gelu_exact(h1 + b1b_ref[0]) * rw1

    y = jax.lax.dot_general(
        h0.astype(jnp.bfloat16), w2a_ref[0].astype(jnp.bfloat16),
        (((1,), (1,)), ((), ())), preferred_element_type=jnp.float32)
    y = y + jax.lax.dot_general(
        h1.astype(jnp.bfloat16), w2b_ref[0].astype(jnp.bfloat16),
        (((1,), (1,)), ((), ())), preferred_element_type=jnp.float32)

    @pl.when(nf == 0)
    def _():
        out_ref[...] = y + rw0 * b2a_ref[0] + rw1 * b2b_ref[0]

    @pl.when(nf > 0)
    def _():
        out_ref[...] += y


def kernel(hidden_states, Wg, W1, b1, W2, b2):
    b, s, d = hidden_states.shape
    n = b * s
    e, f, _ = W1.shape
    x2d = hidden_states.reshape(n, d)

    sel, rw = pl.pallas_call(
        _router_kernel,
        out_shape=(
            jax.ShapeDtypeStruct((_TOPK,), jnp.int32),
            jax.ShapeDtypeStruct((n, 128), jnp.float32),
        ),
        in_specs=[
            pl.BlockSpec((n, d), lambda: (0, 0)),
            pl.BlockSpec((e, d), lambda: (0, 0)),
        ],
        out_specs=(
            pl.BlockSpec(memory_space=pltpu.SMEM),
            pl.BlockSpec((n, 128), lambda: (0, 0)),
        ),
    )(x2d, Wg)

    b1r = b1.reshape(e, 1, f)
    b2r = b2.reshape(e, 1, d)
    nf_steps = f // _FT

    out = pl.pallas_call(
        _expert_kernel,
        grid_spec=pltpu.PrefetchScalarGridSpec(
            num_scalar_prefetch=1,
            grid=(nf_steps,),
            in_specs=[
                pl.BlockSpec((n, d), lambda nf, sel: (0, 0)),
                pl.BlockSpec((n, 128), lambda nf, sel: (0, 0)),
                pl.BlockSpec((1, _FT, d), lambda nf, sel: (sel[0], nf, 0)),
                pl.BlockSpec((1, _FT, d), lambda nf, sel: (sel[1], nf, 0)),
                pl.BlockSpec((1, d, _FT), lambda nf, sel: (sel[0], 0, nf)),
                pl.BlockSpec((1, d, _FT), lambda nf, sel: (sel[1], 0, nf)),
                pl.BlockSpec((1, 1, _FT), lambda nf, sel: (sel[0], 0, nf)),
                pl.BlockSpec((1, 1, _FT), lambda nf, sel: (sel[1], 0, nf)),
                pl.BlockSpec((1, 1, d), lambda nf, sel: (sel[0], 0, 0)),
                pl.BlockSpec((1, 1, d), lambda nf, sel: (sel[1], 0, 0)),
            ],
            out_specs=pl.BlockSpec((n, d), lambda nf, sel: (0, 0)),
        ),
        out_shape=jax.ShapeDtypeStruct((n, d), jnp.float32),
    )(sel, x2d, rw, W1, W1, W2, W2, b1r, b1r, b2r, b2r)

    return out.reshape(b, s, d)


# speedup vs baseline: 3.7247x; 1.0603x over previous
import functools
import math

import jax
import jax.numpy as jnp
from jax.experimental import pallas as pl
from jax.experimental.pallas import tpu as pltpu

_E = 8
_TOPK = 2
_FT = 512

_INV_SQRT2 = 1.0 / math.sqrt(2.0)


def _gelu_exact(h):
    return 0.5 * h * (1.0 + jax.lax.erf(h * _INV_SQRT2))


def _router_kernel(x_ref, wg_ref, sel_ref, rw_ref, x16_ref):
    x16 = x_ref[...].astype(jnp.bfloat16)
    x16_ref[...] = x16
    wg = wg_ref[...]
    logits = jax.lax.dot_general(
        x16, wg.astype(jnp.bfloat16),
        (((1,), (1,)), ((), ())), preferred_element_type=jnp.float32)
    s = jnp.sum(logits, axis=0, keepdims=True)
    eiota = jax.lax.broadcasted_iota(jnp.int32, (1, _E), 1)
    m0 = jnp.max(s)
    i0 = jnp.min(jnp.where(s == m0, eiota, _E))
    s1 = jnp.where(eiota == i0, -jnp.inf, s)
    m1 = jnp.max(s1)
    i1 = jnp.min(jnp.where(s1 == m1, eiota, _E))
    sel_ref[0] = i0
    sel_ref[1] = i1
    l0 = jnp.sum(jnp.where(eiota == i0, logits, 0.0), axis=1, keepdims=True)
    l1 = jnp.sum(jnp.where(eiota == i1, logits, 0.0), axis=1, keepdims=True)
    m = jnp.maximum(l0, l1)
    e0 = jnp.exp(l0 - m)
    e1 = jnp.exp(l1 - m)
    denom = e0 + e1
    r0 = e0 / denom
    r1 = e1 / denom
    lane = jax.lax.broadcasted_iota(jnp.int32, rw_ref.shape, 1)
    rw_ref[...] = jnp.where(lane == 0, r0, jnp.where(lane == 1, r1, 0.0))


def _expert_kernel(sel_ref, x_ref, rw_ref, w1a_ref, w1b_ref, w2a_ref,
                   w2b_ref, b1a_ref, b1b_ref, b2a_ref, b2b_ref, out_ref):
    nf = pl.program_id(0)
    x = x_ref[...]
    rw0 = rw_ref[:, 0:1]
    rw1 = rw_ref[:, 1:2]

    h0 = jax.lax.dot_general(
        x, w1a_ref[0].astype(jnp.bfloat16),
        (((1,), (1,)), ((), ())), preferred_element_type=jnp.float32)
    h0 = _gelu_exact(h0 + b1a_ref[0]) * rw0
    h1 = jax.lax.dot_general(
        x, w1b_ref[0].astype(jnp.bfloat16),
        (((1,), (1,)), ((), ())), preferred_element_type=jnp.float32)
    h1 = _gelu_exact(h1 + b1b_ref[0]) * rw1

    y = jax.lax.dot_general(
        h0.astype(jnp.bfloat16), w2a_ref[0].astype(jnp.bfloat16),
        (((1,), (1,)), ((), ())), preferred_element_type=jnp.float32)
    y = y + jax.lax.dot_general(
        h1.astype(jnp.bfloat16), w2b_ref[0].astype(jnp.bfloat16),
        (((1,), (1,)), ((), ())), preferred_element_type=jnp.float32)

    @pl.when(nf == 0)
    def _():
        out_ref[...] = y + rw0 * b2a_ref[0] + rw1 * b2b_ref[0]

    @pl.when(nf > 0)
    def _():
        out_ref[...] += y


def kernel(hidden_states, Wg, W1, b1, W2, b2):
    b, s, d = hidden_states.shape
    n = b * s
    e, f, _ = W1.shape
    x2d = hidden_states.reshape(n, d)

    sel, rw, x16 = pl.pallas_call(
        _router_kernel,
        out_shape=(
            jax.ShapeDtypeStruct((_TOPK,), jnp.int32),
            jax.ShapeDtypeStruct((n, 128), jnp.float32),
            jax.ShapeDtypeStruct((n, d), jnp.bfloat16),
        ),
        in_specs=[
            pl.BlockSpec((n, d), lambda: (0, 0)),
            pl.BlockSpec((e, d), lambda: (0, 0)),
        ],
        out_specs=(
            pl.BlockSpec(memory_space=pltpu.SMEM),
            pl.BlockSpec((n, 128), lambda: (0, 0)),
            pl.BlockSpec((n, d), lambda: (0, 0)),
        ),
    )(x2d, Wg)

    b1r = b1.reshape(e, 1, f)
    b2r = b2.reshape(e, 1, d)
    nf_steps = f // _FT

    out = pl.pallas_call(
        _expert_kernel,
        grid_spec=pltpu.PrefetchScalarGridSpec(
            num_scalar_prefetch=1,
            grid=(nf_steps,),
            in_specs=[
                pl.BlockSpec((n, d), lambda nf, sel: (0, 0)),
                pl.BlockSpec((n, 128), lambda nf, sel: (0, 0)),
                pl.BlockSpec((1, _FT, d), lambda nf, sel: (sel[0], nf, 0)),
                pl.BlockSpec((1, _FT, d), lambda nf, sel: (sel[1], nf, 0)),
                pl.BlockSpec((1, d, _FT), lambda nf, sel: (sel[0], 0, nf)),
                pl.BlockSpec((1, d, _FT), lambda nf, sel: (sel[1], 0, nf)),
                pl.BlockSpec((1, 1, _FT), lambda nf, sel: (sel[0], 0, nf)),
                pl.BlockSpec((1, 1, _FT), lambda nf, sel: (sel[1], 0, nf)),
                pl.BlockSpec((1, 1, d), lambda nf, sel: (sel[0], 0, 0)),
                pl.BlockSpec((1, 1, d), lambda nf, sel: (sel[1], 0, 0)),
            ],
            out_specs=pl.BlockSpec((n, d), lambda nf, sel: (0, 0)),
        ),
        out_shape=jax.ShapeDtypeStruct((n, d), jnp.float32),
    )(sel, x16, rw, W1, W1, W2, W2, b1r, b1r, b2r, b2r)

    return out.reshape(b, s, d)
